# baseline (speedup 1.0000x reference)
import functools

import numpy as np
import jax
import jax.numpy as jnp
from jax import lax
from jax.experimental import pallas as pl
from jax.experimental.pallas import tpu as pltpu
from jax.experimental.pallas import tpu_sc as plsc

D = 128
L = 20
NVALS = 8
B = 512
NC = 2
NS = 16
NW = NC * NS
CHUNK = 128


def _pe_table() -> np.ndarray:
    half = D // 2
    i = np.arange(half, dtype=np.float64)
    div = np.exp(-(np.log(10000.0)) * (2.0 * i) / D)
    p = np.arange(NVALS, dtype=np.float64)[:, None]
    ang = p * div[None, :]
    tab = np.concatenate([np.sin(ang), np.cos(ang)], axis=-1)
    tab[0] = 0.0
    return tab.astype(np.float32)


def _pe_consts():
    tab = _pe_table()
    j = np.arange(L * NVALS)
    expand = (j[None, :] // NVALS == np.arange(L)[:, None]).astype(np.float32)
    pvals = (j % NVALS).astype(np.float32)
    petab_l = tab[j % NVALS]
    return (jnp.asarray(expand), jnp.asarray(pvals[None, :]),
            jnp.asarray(petab_l))


def _sc_gather(cpw, ids_hbm, table_hbm, out_hbm, idx_v, rows_v, sem):
    w = lax.axis_index("s") * NC + lax.axis_index("c")
    base = w * cpw
    pltpu.sync_copy(ids_hbm.at[pl.ds(base, cpw)], idx_v)
    copies = [
        pltpu.async_copy(table_hbm.at[idx_v.at[j]], rows_v.at[j], sem)
        for j in range(cpw)
    ]
    for c in copies:
        c.wait()
    pltpu.sync_copy(rows_v, out_hbm.at[pl.ds(base, cpw)])


def _sc_gather_call(node_ids, table):
    n = node_ids.shape[0]
    assert n % (NW * CHUNK) == 0
    cpw = n // (NW * CHUNK)
    ids2 = node_ids.reshape(NW * cpw, CHUNK).astype(jnp.int32)
    mesh = plsc.VectorSubcoreMesh(core_axis_name="c", subcore_axis_name="s")
    fn = pl.kernel(
        functools.partial(_sc_gather, cpw),
        mesh=mesh,
        out_type=jax.ShapeDtypeStruct((NW * cpw, CHUNK, D), jnp.float32),
        scratch_types=[
            pltpu.VMEM((cpw, CHUNK), jnp.int32),
            pltpu.VMEM((cpw, CHUNK, D), jnp.float32),
            pltpu.SemaphoreType.DMA,
        ],
    )
    return fn(ids2, table).reshape(n, D)


def _tc_body(pos_ref, gathered_ref, expand_ref, pvals_ref, petab_l_ref,
             out_ref):
    pos = pos_ref[...].astype(jnp.float32)
    posrep = jnp.dot(pos, expand_ref[...],
                     preferred_element_type=jnp.float32)
    onehot = (posrep == pvals_ref[...]).astype(jnp.float32)
    acc = jnp.dot(onehot, petab_l_ref[...],
                  preferred_element_type=jnp.float32)
    norm = jnp.sqrt(jnp.sum(acc * acc, axis=1, keepdims=True))
    acc = acc / (norm + 1e-8)
    out_ref[...] = gathered_ref[...] + acc


def kernel(node_ids, positions, table):
    n = node_ids.shape[0]
    gathered = _sc_gather_call(node_ids, table)
    expand, pvals, petab_l = _pe_consts()
    lv = L * NVALS
    return pl.pallas_call(
        _tc_body,
        grid=(n // B,),
        in_specs=[
            pl.BlockSpec((B, L), lambda i: (i, 0)),
            pl.BlockSpec((B, D), lambda i: (i, 0)),
            pl.BlockSpec((L, lv), lambda i: (0, 0)),
            pl.BlockSpec((1, lv), lambda i: (0, 0)),
            pl.BlockSpec((lv, D), lambda i: (0, 0)),
        ],
        out_specs=pl.BlockSpec((B, D), lambda i: (i, 0)),
        out_shape=jax.ShapeDtypeStruct((n, D), jnp.float32),
    )(positions, jnp.zeros((n, D), jnp.float32), expand, pvals, petab_l
      ) + gathered


# speedup vs baseline: 1.7944x; 1.7944x over previous
import functools

import numpy as np
import jax
import jax.numpy as jnp
from jax import lax
from jax.experimental import pallas as pl
from jax.experimental.pallas import tpu as pltpu
from jax.experimental.pallas import tpu_sc as plsc

D = 128
L = 20
NVALS = 8
B = 512
NC = 2
NS = 16
NW = NC * NS
CHUNK = 128


def _pe_table() -> np.ndarray:
    half = D // 2
    i = np.arange(half, dtype=np.float64)
    div = np.exp(-(np.log(10000.0)) * (2.0 * i) / D)
    p = np.arange(NVALS, dtype=np.float64)[:, None]
    ang = p * div[None, :]
    tab = np.concatenate([np.sin(ang), np.cos(ang)], axis=-1)
    tab[0] = 0.0
    return tab.astype(np.float32)


def _pe_consts():
    tab = _pe_table()
    j = np.arange(L * NVALS)
    expand = (j[None, :] // NVALS == np.arange(L)[:, None]).astype(np.float32)
    pvals = (j % NVALS).astype(np.float32)
    petab_l = tab[j % NVALS]
    return (jnp.asarray(expand), jnp.asarray(pvals[None, :]),
            jnp.asarray(petab_l))


def _sc_gather(cpw, ids_hbm, table_hbm, out_hbm, idx_v, rows_v, sem):
    w = lax.axis_index("s") * NC + lax.axis_index("c")
    base = w * cpw
    pltpu.sync_copy(ids_hbm.at[pl.ds(base, cpw)], idx_v)
    copies = [
        pltpu.async_copy(table_hbm.at[idx_v.at[j]], rows_v.at[j], sem)
        for j in range(cpw)
    ]
    for c in copies:
        c.wait()
    pltpu.sync_copy(rows_v, out_hbm.at[pl.ds(base, cpw)])


def _sc_gather_call(node_ids, table):
    n = node_ids.shape[0]
    assert n % (NW * CHUNK) == 0
    cpw = n // (NW * CHUNK)
    ids2 = node_ids.reshape(NW * cpw, CHUNK).astype(jnp.int32)
    mesh = plsc.VectorSubcoreMesh(core_axis_name="c", subcore_axis_name="s")
    fn = pl.kernel(
        functools.partial(_sc_gather, cpw),
        mesh=mesh,
        out_type=jax.ShapeDtypeStruct((NW * cpw, CHUNK, D), jnp.float32),
        scratch_types=[
            pltpu.VMEM((cpw, CHUNK), jnp.int32),
            pltpu.VMEM((cpw, CHUNK, D), jnp.float32),
            pltpu.SemaphoreType.DMA,
        ],
    )
    return fn(ids2, table).reshape(n, D)


def _pe_acc(pos, expand_ref, pvals_ref, petab_l_ref):
    posrep = jnp.dot(pos, expand_ref[...],
                     preferred_element_type=jnp.float32)
    onehot = (posrep == pvals_ref[...]).astype(jnp.float32)
    acc = jnp.dot(onehot, petab_l_ref[...],
                  preferred_element_type=jnp.float32)
    norm = jnp.sqrt(jnp.sum(acc * acc, axis=1, keepdims=True))
    return acc / (norm + 1e-8)


def _tc_fused_body(ids_ref, pos_ref, tab_ref, expand_ref, pvals_ref,
                   petab_l_ref, out_ref):
    i = pl.program_id(0)
    ids = ids_ref[pl.ds(i * B, B)]
    vocab = tab_ref.shape[0]
    onehot = (ids[:, None] == jax.lax.broadcasted_iota(
        jnp.int32, (1, vocab), 1)).astype(jnp.bfloat16)
    node_vec = jnp.dot(onehot, tab_ref[...],
                       preferred_element_type=jnp.float32)
    acc = _pe_acc(pos_ref[...].astype(jnp.float32), expand_ref, pvals_ref,
                  petab_l_ref)
    out_ref[...] = node_vec + acc


def kernel(node_ids, positions, table):
    n = node_ids.shape[0]
    expand, pvals, petab_l = _pe_consts()
    lv = L * NVALS
    tab_bf16 = table.astype(jnp.bfloat16)
    return pl.pallas_call(
        _tc_fused_body,
        grid=(n // B,),
        in_specs=[
            pl.BlockSpec((n,), lambda i: (0,)),
            pl.BlockSpec((B, L), lambda i: (i, 0)),
            pl.BlockSpec(table.shape, lambda i: (0, 0)),
            pl.BlockSpec((L, lv), lambda i: (0, 0)),
            pl.BlockSpec((1, lv), lambda i: (0, 0)),
            pl.BlockSpec((lv, D), lambda i: (0, 0)),
        ],
        out_specs=pl.BlockSpec((B, D), lambda i: (i, 0)),
        out_shape=jax.ShapeDtypeStruct((n, D), jnp.float32),
    )(node_ids, positions, tab_bf16, expand, pvals, petab_l)


# speedup vs baseline: 2.3055x; 1.2848x over previous
import functools

import numpy as np
import jax
import jax.numpy as jnp
from jax import lax
from jax.experimental import pallas as pl
from jax.experimental.pallas import tpu as pltpu
from jax.experimental.pallas import tpu_sc as plsc

D = 128
L = 20
NVALS = 8
B = 1024
NC = 2
NS = 16
NW = NC * NS
CHUNK = 128


def _pe_table() -> np.ndarray:
    half = D // 2
    i = np.arange(half, dtype=np.float64)
    div = np.exp(-(np.log(10000.0)) * (2.0 * i) / D)
    p = np.arange(NVALS, dtype=np.float64)[:, None]
    ang = p * div[None, :]
    tab = np.concatenate([np.sin(ang), np.cos(ang)], axis=-1)
    tab[0] = 0.0
    return tab.astype(np.float32)


def _pe_consts():
    tab = _pe_table()
    j = np.arange(L * NVALS)
    expand = (j[None, :] // NVALS == np.arange(L)[:, None]).astype(np.float32)
    pvals = (j % NVALS).astype(np.float32)
    petab_l = tab[j % NVALS]
    return (jnp.asarray(expand), jnp.asarray(pvals[None, :]),
            jnp.asarray(petab_l))


def _sc_gather(cpw, ids_hbm, table_hbm, out_hbm, idx_v, rows_v, sem):
    w = lax.axis_index("s") * NC + lax.axis_index("c")
    base = w * cpw
    pltpu.sync_copy(ids_hbm.at[pl.ds(base, cpw)], idx_v)
    copies = [
        pltpu.async_copy(table_hbm.at[idx_v.at[j]], rows_v.at[j], sem)
        for j in range(cpw)
    ]
    for c in copies:
        c.wait()
    pltpu.sync_copy(rows_v, out_hbm.at[pl.ds(base, cpw)])


def _sc_gather_call(node_ids, table):
    n = node_ids.shape[0]
    assert n % (NW * CHUNK) == 0
    cpw = n // (NW * CHUNK)
    ids2 = node_ids.reshape(NW * cpw, CHUNK).astype(jnp.int32)
    mesh = plsc.VectorSubcoreMesh(core_axis_name="c", subcore_axis_name="s")
    fn = pl.kernel(
        functools.partial(_sc_gather, cpw),
        mesh=mesh,
        out_type=jax.ShapeDtypeStruct((NW * cpw, CHUNK, D), jnp.float32),
        scratch_types=[
            pltpu.VMEM((cpw, CHUNK), jnp.int32),
            pltpu.VMEM((cpw, CHUNK, D), jnp.float32),
            pltpu.SemaphoreType.DMA,
        ],
    )
    return fn(ids2, table).reshape(n, D)


def _pe_acc(pos, expand_ref, pvals_ref, petab_l_ref):
    posrep = jnp.dot(pos, expand_ref[...],
                     preferred_element_type=jnp.float32)
    onehot = (posrep == pvals_ref[...]).astype(jnp.float32)
    acc = jnp.dot(onehot, petab_l_ref[...],
                  preferred_element_type=jnp.float32)
    norm = jnp.sqrt(jnp.sum(acc * acc, axis=1, keepdims=True))
    return acc / (norm + 1e-8)


def _tc_fused_body(ids_ref, pos_ref, tab_ref, expand_ref, pvals_ref,
                   petab_l_ref, out_ref):
    i = pl.program_id(0)
    ids = ids_ref[pl.ds(i * B, B)]
    vocab = tab_ref.shape[0]
    onehot = (ids[:, None] == jax.lax.broadcasted_iota(
        jnp.int32, (1, vocab), 1)).astype(jnp.bfloat16)
    node_vec = jnp.dot(onehot, tab_ref[...],
                       preferred_element_type=jnp.float32)
    acc = _pe_acc(pos_ref[...].astype(jnp.float32), expand_ref, pvals_ref,
                  petab_l_ref)
    out_ref[...] = node_vec + acc


def kernel(node_ids, positions, table):
    n = node_ids.shape[0]
    expand, pvals, petab_l = _pe_consts()
    lv = L * NVALS
    tab_bf16 = table.astype(jnp.bfloat16)
    return pl.pallas_call(
        _tc_fused_body,
        grid=(n // B,),
        in_specs=[
            pl.BlockSpec((n,), lambda i: (0,)),
            pl.BlockSpec((B, L), lambda i: (i, 0)),
            pl.BlockSpec(table.shape, lambda i: (0, 0)),
            pl.BlockSpec((L, lv), lambda i: (0, 0)),
            pl.BlockSpec((1, lv), lambda i: (0, 0)),
            pl.BlockSpec((lv, D), lambda i: (0, 0)),
        ],
        out_specs=pl.BlockSpec((B, D), lambda i: (i, 0)),
        out_shape=jax.ShapeDtypeStruct((n, D), jnp.float32),
    )(node_ids, positions, tab_bf16, expand, pvals, petab_l)


# speedup vs baseline: 2.5840x; 1.1208x over previous
import functools

import numpy as np
import jax
import jax.numpy as jnp
from jax import lax
from jax.experimental import pallas as pl
from jax.experimental.pallas import tpu as pltpu
from jax.experimental.pallas import tpu_sc as plsc

D = 128
L = 20
NVALS = 8
B = 2048
NC = 2
NS = 16
NW = NC * NS
CHUNK = 128


def _pe_table() -> np.ndarray:
    half = D // 2
    i = np.arange(half, dtype=np.float64)
    div = np.exp(-(np.log(10000.0)) * (2.0 * i) / D)
    p = np.arange(NVALS, dtype=np.float64)[:, None]
    ang = p * div[None, :]
    tab = np.concatenate([np.sin(ang), np.cos(ang)], axis=-1)
    tab[0] = 0.0
    return tab.astype(np.float32)


def _pe_consts():
    tab = _pe_table()
    j = np.arange(L * NVALS)
    expand = (j[None, :] // NVALS == np.arange(L)[:, None]).astype(np.float32)
    pvals = (j % NVALS).astype(np.float32)
    petab_l = tab[j % NVALS]
    return (jnp.asarray(expand), jnp.asarray(pvals[None, :]),
            jnp.asarray(petab_l))


def _sc_gather(cpw, ids_hbm, table_hbm, out_hbm, idx_v, rows_v, sem):
    w = lax.axis_index("s") * NC + lax.axis_index("c")
    base = w * cpw
    pltpu.sync_copy(ids_hbm.at[pl.ds(base, cpw)], idx_v)
    copies = [
        pltpu.async_copy(table_hbm.at[idx_v.at[j]], rows_v.at[j], sem)
        for j in range(cpw)
    ]
    for c in copies:
        c.wait()
    pltpu.sync_copy(rows_v, out_hbm.at[pl.ds(base, cpw)])


def _sc_gather_call(node_ids, table):
    n = node_ids.shape[0]
    assert n % (NW * CHUNK) == 0
    cpw = n // (NW * CHUNK)
    ids2 = node_ids.reshape(NW * cpw, CHUNK).astype(jnp.int32)
    mesh = plsc.VectorSubcoreMesh(core_axis_name="c", subcore_axis_name="s")
    fn = pl.kernel(
        functools.partial(_sc_gather, cpw),
        mesh=mesh,
        out_type=jax.ShapeDtypeStruct((NW * cpw, CHUNK, D), jnp.float32),
        scratch_types=[
            pltpu.VMEM((cpw, CHUNK), jnp.int32),
            pltpu.VMEM((cpw, CHUNK, D), jnp.float32),
            pltpu.SemaphoreType.DMA,
        ],
    )
    return fn(ids2, table).reshape(n, D)


def _pe_acc(pos, expand_ref, pvals_ref, petab_l_ref):
    posrep = jnp.dot(pos, expand_ref[...],
                     preferred_element_type=jnp.float32)
    onehot = (posrep == pvals_ref[...]).astype(jnp.float32)
    acc = jnp.dot(onehot, petab_l_ref[...],
                  preferred_element_type=jnp.float32)
    norm = jnp.sqrt(jnp.sum(acc * acc, axis=1, keepdims=True))
    return acc / (norm + 1e-8)


def _tc_fused_body(ids_ref, pos_ref, tab_ref, expand_ref, pvals_ref,
                   petab_l_ref, out_ref):
    i = pl.program_id(0)
    ids = ids_ref[pl.ds(i * B, B)]
    vocab = tab_ref.shape[0]
    onehot = (ids[:, None] == jax.lax.broadcasted_iota(
        jnp.int32, (1, vocab), 1)).astype(jnp.bfloat16)
    node_vec = jnp.dot(onehot, tab_ref[...],
                       preferred_element_type=jnp.float32)
    acc = _pe_acc(pos_ref[...].astype(jnp.float32), expand_ref, pvals_ref,
                  petab_l_ref)
    out_ref[...] = node_vec + acc


def kernel(node_ids, positions, table):
    n = node_ids.shape[0]
    expand, pvals, petab_l = _pe_consts()
    lv = L * NVALS
    tab_bf16 = table.astype(jnp.bfloat16)
    return pl.pallas_call(
        _tc_fused_body,
        grid=(n // B,),
        in_specs=[
            pl.BlockSpec((n,), lambda i: (0,)),
            pl.BlockSpec((B, L), lambda i: (i, 0)),
            pl.BlockSpec(table.shape, lambda i: (0, 0)),
            pl.BlockSpec((L, lv), lambda i: (0, 0)),
            pl.BlockSpec((1, lv), lambda i: (0, 0)),
            pl.BlockSpec((lv, D), lambda i: (0, 0)),
        ],
        out_specs=pl.BlockSpec((B, D), lambda i: (i, 0)),
        out_shape=jax.ShapeDtypeStruct((n, D), jnp.float32),
    )(node_ids, positions, tab_bf16, expand, pvals, petab_l)


# speedup vs baseline: 2.6163x; 1.0125x over previous
import functools

import numpy as np
import jax
import jax.numpy as jnp
from jax import lax
from jax.experimental import pallas as pl
from jax.experimental.pallas import tpu as pltpu
from jax.experimental.pallas import tpu_sc as plsc

D = 128
L = 20
NVALS = 8
B = 4096
NC = 2
NS = 16
NW = NC * NS
CHUNK = 128


def _pe_table() -> np.ndarray:
    half = D // 2
    i = np.arange(half, dtype=np.float64)
    div = np.exp(-(np.log(10000.0)) * (2.0 * i) / D)
    p = np.arange(NVALS, dtype=np.float64)[:, None]
    ang = p * div[None, :]
    tab = np.concatenate([np.sin(ang), np.cos(ang)], axis=-1)
    tab[0] = 0.0
    return tab.astype(np.float32)


def _pe_consts():
    tab = _pe_table()
    j = np.arange(L * NVALS)
    expand = (j[None, :] // NVALS == np.arange(L)[:, None]).astype(np.float32)
    pvals = (j % NVALS).astype(np.float32)
    petab_l = tab[j % NVALS]
    return (jnp.asarray(expand), jnp.asarray(pvals[None, :]),
            jnp.asarray(petab_l))


def _sc_gather(cpw, ids_hbm, table_hbm, out_hbm, idx_v, rows_v, sem):
    w = lax.axis_index("s") * NC + lax.axis_index("c")
    base = w * cpw
    pltpu.sync_copy(ids_hbm.at[pl.ds(base, cpw)], idx_v)
    copies = [
        pltpu.async_copy(table_hbm.at[idx_v.at[j]], rows_v.at[j], sem)
        for j in range(cpw)
    ]
    for c in copies:
        c.wait()
    pltpu.sync_copy(rows_v, out_hbm.at[pl.ds(base, cpw)])


def _sc_gather_call(node_ids, table):
    n = node_ids.shape[0]
    assert n % (NW * CHUNK) == 0
    cpw = n // (NW * CHUNK)
    ids2 = node_ids.reshape(NW * cpw, CHUNK).astype(jnp.int32)
    mesh = plsc.VectorSubcoreMesh(core_axis_name="c", subcore_axis_name="s")
    fn = pl.kernel(
        functools.partial(_sc_gather, cpw),
        mesh=mesh,
        out_type=jax.ShapeDtypeStruct((NW * cpw, CHUNK, D), jnp.float32),
        scratch_types=[
            pltpu.VMEM((cpw, CHUNK), jnp.int32),
            pltpu.VMEM((cpw, CHUNK, D), jnp.float32),
            pltpu.SemaphoreType.DMA,
        ],
    )
    return fn(ids2, table).reshape(n, D)


def _pe_acc(pos, expand_ref, pvals_ref, petab_l_ref):
    posrep = jnp.dot(pos, expand_ref[...],
                     preferred_element_type=jnp.float32)
    onehot = (posrep == pvals_ref[...]).astype(jnp.float32)
    acc = jnp.dot(onehot, petab_l_ref[...],
                  preferred_element_type=jnp.float32)
    norm = jnp.sqrt(jnp.sum(acc * acc, axis=1, keepdims=True))
    return acc / (norm + 1e-8)


def _tc_fused_body(ids_ref, pos_ref, tab_ref, expand_ref, pvals_ref,
                   petab_l_ref, out_ref):
    i = pl.program_id(0)
    ids = ids_ref[pl.ds(i * B, B)]
    vocab = tab_ref.shape[0]
    onehot = (ids[:, None] == jax.lax.broadcasted_iota(
        jnp.int32, (1, vocab), 1)).astype(jnp.bfloat16)
    node_vec = jnp.dot(onehot, tab_ref[...],
                       preferred_element_type=jnp.float32)
    acc = _pe_acc(pos_ref[...].astype(jnp.float32), expand_ref, pvals_ref,
                  petab_l_ref)
    out_ref[...] = node_vec + acc


def kernel(node_ids, positions, table):
    n = node_ids.shape[0]
    expand, pvals, petab_l = _pe_consts()
    lv = L * NVALS
    tab_bf16 = table.astype(jnp.bfloat16)
    return pl.pallas_call(
        _tc_fused_body,
        grid=(n // B,),
        in_specs=[
            pl.BlockSpec((n,), lambda i: (0,)),
            pl.BlockSpec((B, L), lambda i: (i, 0)),
            pl.BlockSpec(table.shape, lambda i: (0, 0)),
            pl.BlockSpec((L, lv), lambda i: (0, 0)),
            pl.BlockSpec((1, lv), lambda i: (0, 0)),
            pl.BlockSpec((lv, D), lambda i: (0, 0)),
        ],
        out_specs=pl.BlockSpec((B, D), lambda i: (i, 0)),
        out_shape=jax.ShapeDtypeStruct((n, D), jnp.float32),
    )(node_ids, positions, tab_bf16, expand, pvals, petab_l)
